# baseline (device time: 20630 ns/iter reference)
import jax
import jax.numpy as jnp
from jax import lax
from jax.experimental import pallas as pl
from jax.experimental.pallas import tpu as pltpu

N_DEV = 4
B_LOC = 2
SQ = 128
SKV = 128
HQ = 16
HQ_GRP = 4
DH = 64
D_MODEL = 512
D_QKV = 256

BF16 = jnp.bfloat16
F32 = jnp.float32


def kernel(x, Wq, K_ext, V_ext, Wo):
    my = lax.axis_index("i")
    K_own = jnp.transpose(
        lax.dynamic_slice_in_dim(K_ext, B_LOC * my, B_LOC, axis=0), (0, 2, 1, 3)
    ).astype(BF16)
    V_own = jnp.transpose(
        lax.dynamic_slice_in_dim(V_ext, B_LOC * my, B_LOC, axis=0), (0, 2, 1, 3)
    ).astype(BF16)

    def body(x_ref, wq_ref, k_hbm, v_hbm, wo_ref, out_ref,
             wq16, wo16, wq_comm, wo_comm, k_ref, v_ref, kv_sems,
             wq_send, wq_recv, wo_send, wo_recv):
        my_pos = lax.axis_index("i")
        left = lax.rem(my_pos + N_DEV - 1, N_DEV)
        right = lax.rem(my_pos + 1, N_DEV)
        opp = lax.rem(my_pos + 2, N_DEV)

        k_copy = pltpu.make_async_copy(k_hbm, k_ref, kv_sems.at[0])
        v_copy = pltpu.make_async_copy(v_hbm, v_ref, kv_sems.at[1])
        k_copy.start()
        v_copy.start()

        wq16[...] = wq_ref[...].astype(BF16)
        wo16[...] = wo_ref[...].astype(BF16)

        barrier_sem = pltpu.get_barrier_semaphore()
        for nbr in (left, right, opp):
            pl.semaphore_signal(barrier_sem, inc=1, device_id=(nbr,),
                                device_id_type=pl.DeviceIdType.MESH)
        pl.semaphore_wait(barrier_sem, 3)

        wq_rdmas, wo_rdmas = [], []
        for src, comm, ssem, rsem, out in (
            (wq16, wq_comm, wq_send, wq_recv, wq_rdmas),
            (wo16, wo_comm, wo_send, wo_recv, wo_rdmas),
        ):
            for slot, tgt in ((0, right), (1, left), (2, opp)):
                r = pltpu.make_async_remote_copy(
                    src_ref=src, dst_ref=comm.at[slot],
                    send_sem=ssem.at[slot], recv_sem=rsem.at[slot],
                    device_id=(tgt,), device_id_type=pl.DeviceIdType.MESH)
                r.start()
                out.append(r)

        xm = x_ref[...].reshape(B_LOC * SQ, D_MODEL).astype(BF16)

        def attention(g, wq):
            qm = (jnp.dot(xm, wq, preferred_element_type=F32)
                  * 0.125).astype(BF16)
            g4 = g * HQ_GRP
            out = []
            for b in range(B_LOC):
                qb = qm[b * SQ:(b + 1) * SQ].reshape(SQ, HQ_GRP, DH)
                kb = k_ref[b, pl.ds(g4, HQ_GRP)]
                vb = v_ref[b, pl.ds(g4, HQ_GRP)]
                s4 = lax.dot_general(
                    qb, kb, (((2,), (2,)), ((1,), (0,))),
                    preferred_element_type=F32)
                p4 = jnp.exp(s4)
                r4 = 1.0 / jnp.sum(p4, axis=2, keepdims=True)
                c4 = lax.dot_general(
                    p4.astype(BF16), vb, (((2,), (1,)), ((0,), (0,))),
                    preferred_element_type=F32) * r4
                out.append(c4.astype(BF16))
            return out

        def project(cs, wo):
            ctx = jnp.concatenate(
                [jnp.concatenate([cb[h] for h in range(HQ_GRP)], axis=1)
                 for cb in cs], axis=0)
            return jnp.dot(ctx, wo, preferred_element_type=F32)

        k_copy.wait()
        v_copy.wait()

        acc = project(attention(my_pos, wq16[...]), wo16[...])

        for slot, g in ((0, left), (1, right), (2, opp)):
            wq_rdmas[slot].wait_recv()
            cs = attention(g, wq_comm[slot])
            wo_rdmas[slot].wait_recv()
            acc = acc + project(cs, wo_comm[slot])

        out_ref[...] = acc.reshape(B_LOC, SQ, D_MODEL)

        for r in wq_rdmas + wo_rdmas:
            r.wait_send()

    return pl.pallas_call(
        body,
        out_shape=jax.ShapeDtypeStruct((B_LOC, SQ, D_MODEL), F32),
        in_specs=[
            pl.BlockSpec(memory_space=pltpu.VMEM),
            pl.BlockSpec(memory_space=pltpu.VMEM),
            pl.BlockSpec(memory_space=pl.ANY),
            pl.BlockSpec(memory_space=pl.ANY),
            pl.BlockSpec(memory_space=pltpu.VMEM),
        ],
        out_specs=pl.BlockSpec(memory_space=pltpu.VMEM),
        scratch_shapes=[
            pltpu.VMEM((D_MODEL, D_QKV), BF16),
            pltpu.VMEM((D_QKV, D_MODEL), BF16),
            pltpu.VMEM((3, D_MODEL, D_QKV), BF16),
            pltpu.VMEM((3, D_QKV, D_MODEL), BF16),
            pltpu.VMEM((B_LOC, HQ, SKV, DH), BF16),
            pltpu.VMEM((B_LOC, HQ, SKV, DH), BF16),
            pltpu.SemaphoreType.DMA((2,)),
            pltpu.SemaphoreType.DMA((3,)),
            pltpu.SemaphoreType.DMA((3,)),
            pltpu.SemaphoreType.DMA((3,)),
            pltpu.SemaphoreType.DMA((3,)),
        ],
        compiler_params=pltpu.CompilerParams(collective_id=0),
    )(x, Wq, K_own, V_own, Wo)


# device time: 20475 ns/iter; 1.0076x vs baseline; 1.0076x over previous
import jax
import jax.numpy as jnp
from jax import lax
from jax.experimental import pallas as pl
from jax.experimental.pallas import tpu as pltpu

N_DEV = 4
B_LOC = 2
SQ = 128
SKV = 128
HQ = 16
HQ_GRP = 4
DH = 64
D_MODEL = 512
D_QKV = 256

BF16 = jnp.bfloat16
F32 = jnp.float32


def kernel(x, Wq, K_ext, V_ext, Wo):
    my = lax.axis_index("i")
    K_own = jnp.transpose(
        lax.dynamic_slice_in_dim(K_ext, B_LOC * my, B_LOC, axis=0), (0, 2, 1, 3)
    ).astype(BF16)
    V_own = jnp.transpose(
        lax.dynamic_slice_in_dim(V_ext, B_LOC * my, B_LOC, axis=0), (0, 2, 1, 3)
    ).astype(BF16)

    def body(x_ref, wq_ref, k_ref, v_ref, wo_ref, out_ref,
             wq16, wo16, wq_comm, wo_comm,
             wq_send, wq_recv, wo_send, wo_recv):
        my_pos = lax.axis_index("i")
        left = lax.rem(my_pos + N_DEV - 1, N_DEV)
        right = lax.rem(my_pos + 1, N_DEV)
        opp = lax.rem(my_pos + 2, N_DEV)

        wq16[...] = wq_ref[...].astype(BF16)
        wo16[...] = wo_ref[...].astype(BF16)

        barrier_sem = pltpu.get_barrier_semaphore()
        for nbr in (left, right, opp):
            pl.semaphore_signal(barrier_sem, inc=1, device_id=(nbr,),
                                device_id_type=pl.DeviceIdType.MESH)
        pl.semaphore_wait(barrier_sem, 3)

        wq_rdmas, wo_rdmas = [], []
        for src, comm, ssem, rsem, out in (
            (wq16, wq_comm, wq_send, wq_recv, wq_rdmas),
            (wo16, wo_comm, wo_send, wo_recv, wo_rdmas),
        ):
            for slot, tgt in ((0, right), (1, left), (2, opp)):
                r = pltpu.make_async_remote_copy(
                    src_ref=src, dst_ref=comm.at[slot],
                    send_sem=ssem.at[slot], recv_sem=rsem.at[slot],
                    device_id=(tgt,), device_id_type=pl.DeviceIdType.MESH)
                r.start()
                out.append(r)

        xm = x_ref[...].reshape(B_LOC * SQ, D_MODEL).astype(BF16)

        def attention(g, wq):
            qm = (jnp.dot(xm, wq, preferred_element_type=F32)
                  * 0.125).astype(BF16)
            g4 = g * HQ_GRP
            out = []
            for b in range(B_LOC):
                qb = qm[b * SQ:(b + 1) * SQ].reshape(SQ, HQ_GRP, DH)
                kb = k_ref[b, pl.ds(g4, HQ_GRP)]
                vb = v_ref[b, pl.ds(g4, HQ_GRP)]
                s4 = lax.dot_general(
                    qb, kb, (((2,), (2,)), ((1,), (0,))),
                    preferred_element_type=F32)
                p4 = jnp.exp(s4)
                r4 = 1.0 / jnp.sum(p4, axis=2, keepdims=True)
                c4 = lax.dot_general(
                    p4.astype(BF16), vb, (((2,), (1,)), ((0,), (0,))),
                    preferred_element_type=F32) * r4
                out.append(c4.astype(BF16))
            return out

        def project(cs, wo):
            ctx = jnp.concatenate(
                [jnp.concatenate([cb[h] for h in range(HQ_GRP)], axis=1)
                 for cb in cs], axis=0)
            return jnp.dot(ctx, wo, preferred_element_type=F32)

        acc = project(attention(my_pos, wq16[...]), wo16[...])

        for slot, g in ((0, left), (1, right), (2, opp)):
            wq_rdmas[slot].wait_recv()
            cs = attention(g, wq_comm[slot])
            wo_rdmas[slot].wait_recv()
            acc = acc + project(cs, wo_comm[slot])

        out_ref[...] = acc.reshape(B_LOC, SQ, D_MODEL)

        for r in wq_rdmas + wo_rdmas:
            r.wait_send()

    return pl.pallas_call(
        body,
        out_shape=jax.ShapeDtypeStruct((B_LOC, SQ, D_MODEL), F32),
        in_specs=[pl.BlockSpec(memory_space=pltpu.VMEM)] * 5,
        out_specs=pl.BlockSpec(memory_space=pltpu.VMEM),
        scratch_shapes=[
            pltpu.VMEM((D_MODEL, D_QKV), BF16),
            pltpu.VMEM((D_QKV, D_MODEL), BF16),
            pltpu.VMEM((3, D_MODEL, D_QKV), BF16),
            pltpu.VMEM((3, D_QKV, D_MODEL), BF16),
            pltpu.SemaphoreType.DMA((3,)),
            pltpu.SemaphoreType.DMA((3,)),
            pltpu.SemaphoreType.DMA((3,)),
            pltpu.SemaphoreType.DMA((3,)),
        ],
        compiler_params=pltpu.CompilerParams(collective_id=0),
    )(x, Wq, K_own, V_own, Wo)
